# baseline (speedup 1.0000x reference)
import functools

import jax
import jax.numpy as jnp
from jax import lax
from jax.experimental import pallas as pl
from jax.experimental.pallas import tpu as pltpu
from jax.experimental.pallas import tpu_sc as plsc

N = 10000
E = 160000
D = 256
H = 128
NC = 2
NS = 16
K = 64
CHUNKS = 4 * (-(-E // (NS * K * 4)))
EPT = CHUNKS * K
E_PAD = NS * EPT
ACC_ROWS = 10240
ZPT = ACC_ROWS // NS
OPT = 624
TAIL = N - NS * OPT
BI = 2000
GI = N // BI


def _vmesh():
    return plsc.VectorSubcoreMesh(core_axis_name="c", subcore_axis_name="s")


def _sc_deg(dst_pad, zeros_big, ones_big):
    NB = 4

    @functools.partial(
        pl.kernel,
        out_type=jax.ShapeDtypeStruct((N, H), jnp.float32),
        mesh=_vmesh(),
        scratch_types=[
            pltpu.VMEM((CHUNKS, K), jnp.int32),
            pltpu.VMEM((K, H), jnp.float32),
            pltpu.VMEM_SHARED((ACC_ROWS, H), jnp.float32),
            [pltpu.SemaphoreType.DMA] * NB,
        ],
    )
    def k(dst_hbm, z_hbm, o_hbm, out_hbm, dst_v, ones_v, acc, sem_s):
        c = lax.axis_index("c")
        s = lax.axis_index("s")
        pltpu.sync_copy(dst_hbm.at[pl.ds(s * CHUNKS, CHUNKS)], dst_v)
        pltpu.sync_copy(z_hbm.at[pl.ds(s * ZPT, ZPT)], acc.at[pl.ds(s * ZPT, ZPT)])
        pltpu.sync_copy(o_hbm, ones_v)
        plsc.subcore_barrier()

        def body(t, carry):
            for b in range(NB):
                g = NB * t + b

                @pl.when(g >= NB)
                def _():
                    pltpu.make_async_copy(ones_v, acc.at[dst_v.at[0]],
                                          sem_s[b]).wait()

                pltpu.async_copy(ones_v, acc.at[dst_v.at[g]], sem_s[b],
                                 add=True)
            return carry

        lax.fori_loop(0, CHUNKS // NB, body, 0)
        for b in range(NB):
            pltpu.make_async_copy(ones_v, acc.at[dst_v.at[0]], sem_s[b]).wait()
        plsc.subcore_barrier()

        @pl.when(c == 0)
        def _():
            pltpu.sync_copy(acc.at[pl.ds(s * OPT, OPT)],
                            out_hbm.at[pl.ds(s * OPT, OPT)])

        @pl.when((c == 0) & (s == NS - 1))
        def _():
            pltpu.sync_copy(acc.at[pl.ds(NS * OPT, TAIL)],
                            out_hbm.at[pl.ds(NS * OPT, TAIL)])

    return k(dst_pad, zeros_big, ones_big)


def _sc_scatter(m_flat, src2, dst_pad, zeros_big):
    NB = 4
    LOOK = NB - 1

    @functools.partial(
        pl.kernel,
        out_type=jax.ShapeDtypeStruct((NC * N, H), jnp.float32),
        mesh=_vmesh(),
        scratch_types=[
            pltpu.VMEM((CHUNKS * K,), jnp.int32),
            [pltpu.VMEM((K,), jnp.int32)] * NB,
            [pltpu.VMEM((K, H), jnp.float32)] * NB,
            pltpu.VMEM_SHARED((ACC_ROWS, H), jnp.float32),
            [pltpu.SemaphoreType.DMA] * NB,
            [pltpu.SemaphoreType.DMA] * NB,
            [pltpu.SemaphoreType.DMA] * NB,
        ],
    )
    def k(m_hbm, src_hbm, dst_hbm, z_hbm, out_hbm, src_v, dst_v, rows_v, acc,
          sem_g, sem_d, sem_s):
        c = lax.axis_index("c")
        s = lax.axis_index("s")
        pltpu.sync_copy(src_hbm.at[pl.ds(c * E_PAD + s * EPT, EPT)], src_v)
        pltpu.sync_copy(z_hbm.at[pl.ds(s * ZPT, ZPT)], acc.at[pl.ds(s * ZPT, ZPT)])
        for j in range(LOOK):
            pltpu.async_copy(dst_hbm.at[pl.ds(s * EPT + j * K, K)],
                             dst_v[j], sem_d[j])
            pltpu.async_copy(m_hbm.at[src_v.at[pl.ds(j * K, K)]], rows_v[j], sem_g[j])
        plsc.subcore_barrier()

        def body(t, carry):
            for b in range(NB):
                g = NB * t + b
                bf = (b + LOOK) % NB

                @pl.when(g + LOOK < CHUNKS)
                def _():
                    @pl.when(g >= 1)
                    def _():
                        pltpu.make_async_copy(
                            rows_v[bf], acc.at[dst_v[bf]], sem_s[bf]).wait()

                    pltpu.async_copy(
                        dst_hbm.at[pl.ds(s * EPT + (g + LOOK) * K, K)],
                        dst_v[bf], sem_d[bf])
                    pltpu.async_copy(m_hbm.at[src_v.at[pl.ds((g + LOOK) * K, K)]],
                                     rows_v[bf], sem_g[bf])

                pltpu.make_async_copy(m_hbm.at[src_v.at[pl.ds(g * K, K)]],
                                      rows_v[b], sem_g[b]).wait()
                pltpu.make_async_copy(dst_hbm.at[pl.ds(s * EPT, K)],
                                      dst_v[b], sem_d[b]).wait()
                pltpu.async_copy(rows_v[b], acc.at[dst_v[b]], sem_s[b],
                                 add=True)
            return carry

        lax.fori_loop(0, CHUNKS // NB, body, 0)
        for b in range(NB):
            pltpu.make_async_copy(rows_v[b], acc.at[dst_v[b]], sem_s[b]).wait()
        plsc.subcore_barrier()
        pltpu.sync_copy(acc.at[pl.ds(s * OPT, OPT)],
                        out_hbm.at[pl.ds(c * N + s * OPT, OPT)])

        @pl.when(s == NS - 1)
        def _():
            pltpu.sync_copy(acc.at[pl.ds(NS * OPT, TAIL)],
                            out_hbm.at[pl.ds(c * N + NS * OPT, TAIL)])

    return k(m_flat, src2, dst_pad, zeros_big)


def _dis(deg_ref):
    deg = deg_ref[:, :1]
    return jnp.where(deg > 0, lax.rsqrt(jnp.maximum(deg, 1.0)), 0.0)


def _pre_body(x_ref, w_ref, deg_ref, o_ref):
    dis = _dis(deg_ref)
    m = jnp.dot(x_ref[...], w_ref[...], preferred_element_type=jnp.float32)
    o_ref[...] = m * dis


def _tc_pre(x, w, deg16):
    return pl.pallas_call(
        _pre_body,
        grid=(NC, GI),
        in_specs=[
            pl.BlockSpec((BI, D), lambda c, i: (i, 0)),
            pl.BlockSpec((D, H), lambda c, i: (0, c)),
            pl.BlockSpec((BI, H), lambda c, i: (i, 0)),
        ],
        out_specs=pl.BlockSpec((BI, H), lambda c, i: (c * GI + i, 0)),
        out_shape=jax.ShapeDtypeStruct((NC * N, H), jnp.float32),
    )(x, w, deg16)


def _mid_body(sa_ref, sb_ref, deg_ref, b_ref, w_ref, o_ref):
    dis = _dis(deg_ref)
    h = jnp.concatenate([sa_ref[...], sb_ref[...]], axis=1) * dis + b_ref[...]
    m = jnp.dot(h, w_ref[...], preferred_element_type=jnp.float32)
    o_ref[...] = m * dis


def _tc_mid(s_flat, deg16, b2d, w):
    return pl.pallas_call(
        _mid_body,
        grid=(NC, GI),
        in_specs=[
            pl.BlockSpec((BI, H), lambda c, i: (i, 0)),
            pl.BlockSpec((BI, H), lambda c, i: (GI + i, 0)),
            pl.BlockSpec((BI, H), lambda c, i: (i, 0)),
            pl.BlockSpec((1, D), lambda c, i: (0, 0)),
            pl.BlockSpec((D, H), lambda c, i: (0, c)),
        ],
        out_specs=pl.BlockSpec((BI, H), lambda c, i: (c * GI + i, 0)),
        out_shape=jax.ShapeDtypeStruct((NC * N, H), jnp.float32),
    )(s_flat, s_flat, deg16, b2d, w)


def _final_body(sa_ref, sb_ref, deg_ref, b_ref, o_ref):
    dis = _dis(deg_ref)
    o_ref[...] = (jnp.concatenate([sa_ref[...], sb_ref[...]], axis=1) * dis
                  + b_ref[...])


def _tc_final(s_flat, deg16, b2d):
    return pl.pallas_call(
        _final_body,
        grid=(GI,),
        in_specs=[
            pl.BlockSpec((BI, H), lambda i: (i, 0)),
            pl.BlockSpec((BI, H), lambda i: (GI + i, 0)),
            pl.BlockSpec((BI, H), lambda i: (i, 0)),
            pl.BlockSpec((1, D), lambda i: (0, 0)),
        ],
        out_specs=pl.BlockSpec((BI, D), lambda i: (i, 0)),
        out_shape=jax.ShapeDtypeStruct((N, D), jnp.float32),
    )(s_flat, s_flat, deg16, b2d)


def kernel(x, edge_index, W0, b0, W1, b1, W2, b2):
    src = edge_index[0].astype(jnp.int32)
    dst = edge_index[1].astype(jnp.int32)
    pad = E_PAD - E
    src_p = jnp.concatenate([src, jnp.zeros((pad,), jnp.int32)])
    dst_p = jnp.concatenate([dst, jnp.full((pad,), N, jnp.int32)])
    src2 = jnp.concatenate([src_p, src_p + N])
    dst_2d = dst_p.reshape(NS * CHUNKS, K)
    zeros_big = jnp.zeros((ACC_ROWS, H), jnp.float32)
    ones_big = jnp.ones((K, H), jnp.float32)

    deg16 = _sc_deg(dst_2d, zeros_big, ones_big)
    m = _tc_pre(x, W0, deg16)
    s = _sc_scatter(m, src2, dst_p, zeros_big)
    m = _tc_mid(s, deg16, b0.reshape(1, D), W1)
    s = _sc_scatter(m, src2, dst_p, zeros_big)
    m = _tc_mid(s, deg16, b1.reshape(1, D), W2)
    s = _sc_scatter(m, src2, dst_p, zeros_big)
    return _tc_final(s, deg16, b2.reshape(1, D))


# speedup vs baseline: 1.0142x; 1.0142x over previous
import functools

import jax
import jax.numpy as jnp
from jax import lax
from jax.experimental import pallas as pl
from jax.experimental.pallas import tpu as pltpu
from jax.experimental.pallas import tpu_sc as plsc

N = 10000
E = 160000
D = 256
H = 128
NC = 2
NS = 16
K = 64
CHUNKS = 4 * (-(-E // (NS * K * 4)))
EPT = CHUNKS * K
E_PAD = NS * EPT
ACC_ROWS = 10240
ZPT = ACC_ROWS // NS
OPT = 624
TAIL = N - NS * OPT
BI = 1000
GI = N // BI


def _vmesh():
    return plsc.VectorSubcoreMesh(core_axis_name="c", subcore_axis_name="s")


def _sc_deg(dst_pad, zeros_big, ones_big):
    NB = 4

    @functools.partial(
        pl.kernel,
        out_type=jax.ShapeDtypeStruct((N, H), jnp.float32),
        mesh=_vmesh(),
        scratch_types=[
            pltpu.VMEM((CHUNKS, K), jnp.int32),
            pltpu.VMEM((K, H), jnp.float32),
            pltpu.VMEM_SHARED((ACC_ROWS, H), jnp.float32),
            [pltpu.SemaphoreType.DMA] * NB,
        ],
    )
    def k(dst_hbm, z_hbm, o_hbm, out_hbm, dst_v, ones_v, acc, sem_s):
        c = lax.axis_index("c")
        s = lax.axis_index("s")
        pltpu.sync_copy(dst_hbm.at[pl.ds(s * CHUNKS, CHUNKS)], dst_v)
        pltpu.sync_copy(z_hbm.at[pl.ds(s * ZPT, ZPT)], acc.at[pl.ds(s * ZPT, ZPT)])
        pltpu.sync_copy(o_hbm, ones_v)
        plsc.subcore_barrier()

        def body(t, carry):
            for b in range(NB):
                g = NB * t + b

                @pl.when(g >= NB)
                def _():
                    pltpu.make_async_copy(ones_v, acc.at[dst_v.at[0]],
                                          sem_s[b]).wait()

                pltpu.async_copy(ones_v, acc.at[dst_v.at[g]], sem_s[b],
                                 add=True)
            return carry

        lax.fori_loop(0, CHUNKS // NB, body, 0)
        for b in range(NB):
            pltpu.make_async_copy(ones_v, acc.at[dst_v.at[0]], sem_s[b]).wait()
        plsc.subcore_barrier()

        @pl.when(c == 0)
        def _():
            pltpu.sync_copy(acc.at[pl.ds(s * OPT, OPT)],
                            out_hbm.at[pl.ds(s * OPT, OPT)])

        @pl.when((c == 0) & (s == NS - 1))
        def _():
            pltpu.sync_copy(acc.at[pl.ds(NS * OPT, TAIL)],
                            out_hbm.at[pl.ds(NS * OPT, TAIL)])

    return k(dst_pad, zeros_big, ones_big)


def _sc_scatter(m_flat, src2, dst_pad, zeros_big):
    NB = 4
    LOOK = NB - 1

    @functools.partial(
        pl.kernel,
        out_type=jax.ShapeDtypeStruct((NC * N, H), jnp.float32),
        mesh=_vmesh(),
        scratch_types=[
            pltpu.VMEM((CHUNKS * K,), jnp.int32),
            [pltpu.VMEM((K,), jnp.int32)] * NB,
            [pltpu.VMEM((K, H), jnp.float32)] * NB,
            pltpu.VMEM_SHARED((ACC_ROWS, H), jnp.float32),
            [pltpu.SemaphoreType.DMA] * NB,
            [pltpu.SemaphoreType.DMA] * NB,
            [pltpu.SemaphoreType.DMA] * NB,
        ],
    )
    def k(m_hbm, src_hbm, dst_hbm, z_hbm, out_hbm, src_v, dst_v, rows_v, acc,
          sem_g, sem_d, sem_s):
        c = lax.axis_index("c")
        s = lax.axis_index("s")
        pltpu.sync_copy(src_hbm.at[pl.ds(c * E_PAD + s * EPT, EPT)], src_v)
        pltpu.sync_copy(z_hbm.at[pl.ds(s * ZPT, ZPT)], acc.at[pl.ds(s * ZPT, ZPT)])
        for j in range(LOOK):
            pltpu.async_copy(dst_hbm.at[pl.ds(s * EPT + j * K, K)],
                             dst_v[j], sem_d[j])
            pltpu.async_copy(m_hbm.at[src_v.at[pl.ds(j * K, K)]], rows_v[j], sem_g[j])
        plsc.subcore_barrier()

        def body(t, carry):
            for b in range(NB):
                g = NB * t + b
                bf = (b + LOOK) % NB

                @pl.when(g + LOOK < CHUNKS)
                def _():
                    @pl.when(g >= 1)
                    def _():
                        pltpu.make_async_copy(
                            rows_v[bf], acc.at[dst_v[bf]], sem_s[bf]).wait()

                    pltpu.async_copy(
                        dst_hbm.at[pl.ds(s * EPT + (g + LOOK) * K, K)],
                        dst_v[bf], sem_d[bf])
                    pltpu.async_copy(m_hbm.at[src_v.at[pl.ds((g + LOOK) * K, K)]],
                                     rows_v[bf], sem_g[bf])

                pltpu.make_async_copy(m_hbm.at[src_v.at[pl.ds(g * K, K)]],
                                      rows_v[b], sem_g[b]).wait()
                pltpu.make_async_copy(dst_hbm.at[pl.ds(s * EPT, K)],
                                      dst_v[b], sem_d[b]).wait()
                pltpu.async_copy(rows_v[b], acc.at[dst_v[b]], sem_s[b],
                                 add=True)
            return carry

        lax.fori_loop(0, CHUNKS // NB, body, 0)
        for b in range(NB):
            pltpu.make_async_copy(rows_v[b], acc.at[dst_v[b]], sem_s[b]).wait()
        plsc.subcore_barrier()
        pltpu.sync_copy(acc.at[pl.ds(s * OPT, OPT)],
                        out_hbm.at[pl.ds(c * N + s * OPT, OPT)])

        @pl.when(s == NS - 1)
        def _():
            pltpu.sync_copy(acc.at[pl.ds(NS * OPT, TAIL)],
                            out_hbm.at[pl.ds(c * N + NS * OPT, TAIL)])

    return k(m_flat, src2, dst_pad, zeros_big)


def _dis(deg_ref):
    deg = deg_ref[:, :1]
    return jnp.where(deg > 0, lax.rsqrt(jnp.maximum(deg, 1.0)), 0.0)


def _pre_body(x_ref, w_ref, deg_ref, o_ref):
    dis = _dis(deg_ref)
    m = jnp.dot(x_ref[...], w_ref[...], preferred_element_type=jnp.float32)
    o_ref[...] = m * dis


def _tc_pre(x, w, deg16):
    return pl.pallas_call(
        _pre_body,
        grid=(NC, GI),
        in_specs=[
            pl.BlockSpec((BI, D), lambda c, i: (i, 0)),
            pl.BlockSpec((D, H), lambda c, i: (0, c)),
            pl.BlockSpec((BI, H), lambda c, i: (i, 0)),
        ],
        out_specs=pl.BlockSpec((BI, H), lambda c, i: (c * GI + i, 0)),
        out_shape=jax.ShapeDtypeStruct((NC * N, H), jnp.float32),
    )(x, w, deg16)


def _mid_body(sa_ref, sb_ref, deg_ref, b_ref, w_ref, o_ref):
    dis = _dis(deg_ref)
    h = jnp.concatenate([sa_ref[...], sb_ref[...]], axis=1) * dis + b_ref[...]
    m = jnp.dot(h, w_ref[...], preferred_element_type=jnp.float32)
    o_ref[...] = m * dis


def _tc_mid(s_flat, deg16, b2d, w):
    return pl.pallas_call(
        _mid_body,
        grid=(NC, GI),
        in_specs=[
            pl.BlockSpec((BI, H), lambda c, i: (i, 0)),
            pl.BlockSpec((BI, H), lambda c, i: (GI + i, 0)),
            pl.BlockSpec((BI, H), lambda c, i: (i, 0)),
            pl.BlockSpec((1, D), lambda c, i: (0, 0)),
            pl.BlockSpec((D, H), lambda c, i: (0, c)),
        ],
        out_specs=pl.BlockSpec((BI, H), lambda c, i: (c * GI + i, 0)),
        out_shape=jax.ShapeDtypeStruct((NC * N, H), jnp.float32),
    )(s_flat, s_flat, deg16, b2d, w)


def _final_body(sa_ref, sb_ref, deg_ref, b_ref, o_ref):
    dis = _dis(deg_ref)
    o_ref[...] = (jnp.concatenate([sa_ref[...], sb_ref[...]], axis=1) * dis
                  + b_ref[...])


def _tc_final(s_flat, deg16, b2d):
    return pl.pallas_call(
        _final_body,
        grid=(GI,),
        in_specs=[
            pl.BlockSpec((BI, H), lambda i: (i, 0)),
            pl.BlockSpec((BI, H), lambda i: (GI + i, 0)),
            pl.BlockSpec((BI, H), lambda i: (i, 0)),
            pl.BlockSpec((1, D), lambda i: (0, 0)),
        ],
        out_specs=pl.BlockSpec((BI, D), lambda i: (i, 0)),
        out_shape=jax.ShapeDtypeStruct((N, D), jnp.float32),
    )(s_flat, s_flat, deg16, b2d)


def kernel(x, edge_index, W0, b0, W1, b1, W2, b2):
    src = edge_index[0].astype(jnp.int32)
    dst = edge_index[1].astype(jnp.int32)
    pad = E_PAD - E
    src_p = jnp.concatenate([src, jnp.zeros((pad,), jnp.int32)])
    dst_p = jnp.concatenate([dst, jnp.full((pad,), N, jnp.int32)])
    src2 = jnp.concatenate([src_p, src_p + N])
    dst_2d = dst_p.reshape(NS * CHUNKS, K)
    zeros_big = jnp.zeros((ACC_ROWS, H), jnp.float32)
    ones_big = jnp.ones((K, H), jnp.float32)

    deg16 = _sc_deg(dst_2d, zeros_big, ones_big)
    m = _tc_pre(x, W0, deg16)
    s = _sc_scatter(m, src2, dst_p, zeros_big)
    m = _tc_mid(s, deg16, b0.reshape(1, D), W1)
    s = _sc_scatter(m, src2, dst_p, zeros_big)
    m = _tc_mid(s, deg16, b1.reshape(1, D), W2)
    s = _sc_scatter(m, src2, dst_p, zeros_big)
    return _tc_final(s, deg16, b2.reshape(1, D))
